# baseline (speedup 1.0000x reference)
import functools

import jax
import jax.numpy as jnp
from jax import lax
from jax.experimental import pallas as pl
from jax.experimental.pallas import tpu as pltpu
from jax.experimental.pallas import tpu_sc as plsc

N = 10000
E = 320000
D_IN = 128
HIDDEN = 128
NUM_CLASSES = 10
NG = 64

NC = 2
NS = 16
TILES = NC * NS
RPT = 80
ROWS = TILES * RPT
NPAD = 10240
SPT = 3
CH = RPT + SPT
CN = NG * N
NW = 10112


def _sc_mesh():
    return plsc.VectorSubcoreMesh(core_axis_name="c", subcore_axis_name="s")


def _sc_degree(er3):
    @functools.partial(
        pl.kernel,
        mesh=_sc_mesh(),
        compiler_params=pltpu.CompilerParams(needs_layout_passes=False),
        out_type=jax.ShapeDtypeStruct((NC * NPAD,), jnp.float32),
        scratch_types=[
            pltpu.VMEM((RPT, 128), jnp.int32),
            pltpu.VMEM((128,), jnp.float32),
            pltpu.VMEM((640,), jnp.float32),
            pltpu.VMEM_SHARED((NPAD,), jnp.float32),
            pltpu.SemaphoreType.DMA,
            pltpu.SemaphoreType.DMA,
        ],
    )
    def k(er_hbm, out_hbm, dst_v, ones_v, zero_v, deg_sh, sem_in, sem_s):
        c = lax.axis_index("c")
        s = lax.axis_index("s")
        wid = s * NC + c
        nrows = jnp.clip(E // 128 - wid * RPT, 0, RPT)

        pltpu.async_copy(er_hbm.at[1, pl.ds(wid * RPT, RPT), :], dst_v,
                         sem_in)

        def fbody(i, _):
            ones_v[pl.ds(i * 16, 16)] = jnp.full((16,), 1.0, jnp.float32)
            zero_v[pl.ds(i * 16, 16)] = jnp.zeros((16,), jnp.float32)
            return 0

        lax.fori_loop(0, 8, fbody, 0)

        def zbody(i, _):
            zero_v[pl.ds(128 + i * 16, 16)] = jnp.zeros((16,), jnp.float32)
            return 0

        lax.fori_loop(0, (640 - 128) // 16, zbody, 0)
        pltpu.sync_copy(zero_v, deg_sh.at[pl.ds(s * 640, 640)])
        pltpu.make_async_copy(er_hbm.at[1, pl.ds(wid * RPT, RPT), :], dst_v,
                              sem_in).wait()
        plsc.subcore_barrier()

        def sbody(j, _):
            pltpu.async_copy(ones_v, deg_sh.at[dst_v.at[j]], sem_s, add=True)
            return 0

        lax.fori_loop(0, nrows, sbody, 0)

        def dbody(j, _):
            pltpu.make_async_copy(ones_v, deg_sh.at[dst_v.at[0]],
                                  sem_s).wait()
            return 0

        lax.fori_loop(0, nrows, dbody, 0)
        plsc.subcore_barrier()
        pltpu.sync_copy(deg_sh.at[pl.ds(s * 640, 640)], zero_v)
        pltpu.sync_copy(zero_v, out_hbm.at[pl.ds(c * NPAD + s * 640, 640)])

    return k(er3)


def _tc_dinv(deg_part):
    def body(d_ref, o_ref):
        deg = d_ref[0] + d_ref[1] + 1.0
        dinv = lax.rsqrt(deg)
        r = lax.broadcasted_iota(jnp.int32, (NPAD // 128, 128), 0)
        l = lax.broadcasted_iota(jnp.int32, (NPAD // 128, 128), 1)
        o_ref[...] = jnp.where(r * 128 + l < N, dinv, 0.0)

    return pl.pallas_call(
        body,
        out_shape=jax.ShapeDtypeStruct((NPAD // 128, 128), jnp.float32),
    )(deg_part)


def _sc_coef(er3, dinv_p, batch_p):
    ZS = CN // NS

    @functools.partial(
        pl.kernel,
        mesh=_sc_mesh(),
        compiler_params=pltpu.CompilerParams(needs_layout_passes=False),
        out_type=jax.ShapeDtypeStruct((NC * CN,), jnp.float32),
        scratch_types=[
            pltpu.VMEM((RPT, 128), jnp.int32),
            pltpu.VMEM((RPT, 128), jnp.int32),
            pltpu.VMEM((NPAD,), jnp.float32),
            pltpu.VMEM((NPAD,), jnp.int32),
            pltpu.VMEM((CH, 128), jnp.int32),
            pltpu.VMEM((CH, 128), jnp.float32),
            pltpu.VMEM((8000,), jnp.float32),
            pltpu.VMEM((8000,), jnp.float32),
            pltpu.VMEM_SHARED((CN,), jnp.float32),
            pltpu.SemaphoreType.DMA,
            pltpu.SemaphoreType.DMA,
            pltpu.SemaphoreType.DMA,
            pltpu.SemaphoreType.DMA,
            pltpu.SemaphoreType.DMA,
            pltpu.SemaphoreType.DMA,
        ],
    )
    def k(er_hbm, dinv_hbm, batch_hbm, out_hbm,
          src_v, dst_v, dinv_v, batch_v, idx_v, val_v, buf0, buf1,
          c_sh, sem_in, sem_s, sem_z, sem_a1, sem_b0, sem_b1):
        c = lax.axis_index("c")
        s = lax.axis_index("s")
        wid = s * NC + c
        nrows = jnp.clip(E // 128 - wid * RPT, 0, RPT)
        base = wid * (SPT * 128)
        nself = jnp.where(base < N, SPT, 0)

        pltpu.async_copy(er_hbm.at[0, pl.ds(wid * RPT, RPT), :], src_v,
                         sem_in)
        pltpu.async_copy(er_hbm.at[1, pl.ds(wid * RPT, RPT), :], dst_v,
                         sem_in)
        pltpu.async_copy(dinv_hbm, dinv_v, sem_in)
        pltpu.async_copy(batch_hbm, batch_v, sem_in)

        def zbody(i, _):
            buf0[pl.ds(i * 16, 16)] = jnp.zeros((16,), jnp.float32)
            return 0

        lax.fori_loop(0, 8000 // 16, zbody, 0)

        def zcopy(i, _):
            pltpu.async_copy(buf0, c_sh.at[pl.ds(s * ZS + i * 8000, 8000)],
                             sem_z)
            return 0

        lax.fori_loop(0, ZS // 8000, zcopy, 0)

        pltpu.make_async_copy(er_hbm.at[0, pl.ds(wid * RPT, RPT), :], src_v,
                              sem_in).wait()
        pltpu.make_async_copy(er_hbm.at[1, pl.ds(wid * RPT, RPT), :], dst_v,
                              sem_in).wait()
        pltpu.make_async_copy(dinv_hbm, dinv_v, sem_in).wait()
        pltpu.make_async_copy(batch_hbm, batch_v, sem_in).wait()

        def cbody(j, _):
            for kk in range(8):
                sl = pl.ds(kk * 16, 16)
                d16 = dst_v[j, sl]
                s16 = src_v[j, sl]
                b16 = plsc.load_gather(batch_v, [d16])
                dv = plsc.load_gather(dinv_v, [d16])
                val_v[j, sl] = dv
                idx_v[j, sl] = b16 * N + s16
            return 0

        lax.fori_loop(0, nrows, cbody, 0)

        @pl.when(base < N)
        def _():
            for jj in range(SPT):
                for kk in range(8):
                    sl = pl.ds(kk * 16, 16)
                    i16 = (base + jj * 128 + kk * 16
                           + lax.broadcasted_iota(jnp.int32, (16,), 0))
                    i16 = jnp.where(i16 >= NPAD, i16 - 128, i16)
                    b16 = plsc.load_gather(batch_v, [i16])
                    dv = plsc.load_gather(dinv_v, [i16])
                    val_v[RPT + jj, sl] = dv
                    idx_v[RPT + jj, sl] = b16 * N + i16

        def zdrain(i, _):
            pltpu.make_async_copy(buf0, c_sh.at[pl.ds(s * ZS, 8000)],
                                  sem_z).wait()
            return 0

        lax.fori_loop(0, ZS // 8000, zdrain, 0)
        plsc.subcore_barrier()

        def fbody(j, _):
            pltpu.async_copy(val_v.at[j], c_sh.at[idx_v.at[j]], sem_s,
                             add=True)
            return 0

        lax.fori_loop(0, nrows + nself, fbody, 0)

        def dbody(j, _):
            pltpu.make_async_copy(val_v.at[0], c_sh.at[idx_v.at[0]],
                                  sem_s).wait()
            return 0

        lax.fori_loop(0, nrows + nself, dbody, 0)
        plsc.subcore_barrier()

        bufs = (buf0, buf1)
        sa = (sem_in, sem_a1)
        sb = (sem_b0, sem_b1)

        def rd(i):
            return (c_sh.at[pl.ds(s * ZS + i * 8000, 8000)], bufs[i % 2],
                    sa[i % 2])

        def wr(i):
            return (bufs[i % 2],
                    out_hbm.at[pl.ds(c * CN + s * ZS + i * 8000, 8000)],
                    sb[i % 2])

        nchunk = ZS // 8000
        pltpu.async_copy(*rd(0))
        pltpu.async_copy(*rd(1))
        for i in range(nchunk):
            pltpu.make_async_copy(*rd(i)).wait()
            pltpu.async_copy(*wr(i))
            if i + 2 < nchunk:
                pltpu.make_async_copy(*wr(i)).wait()
                pltpu.async_copy(*rd(i + 2))
        pltpu.make_async_copy(*wr(nchunk - 2)).wait()
        pltpu.make_async_copy(*wr(nchunk - 1)).wait()

    return k(er3, dinv_p, batch_p)


def _tc_counts(batch_row):
    def body(b_ref, o_ref):
        g = lax.broadcasted_iota(jnp.int32, (NG, N), 0)
        o_ref[...] = jnp.sum(jnp.where(b_ref[...] == g, 1.0, 0.0), axis=1,
                             keepdims=True)

    return pl.pallas_call(
        body,
        out_shape=jax.ShapeDtypeStruct((NG, 1), jnp.float32),
    )(batch_row)


def _tc_head(cp, x, dinv_col, cnt, Wc, bc, Wl, bl):
    def body(cp_ref, x_ref, dinv_ref, cnt_ref, wc_ref, bc_ref, wl_ref, bl_ref,
             o_ref):
        C = cp_ref[0] + cp_ref[1]
        xd = x_ref[...] * dinv_ref[...]
        z = jnp.dot(C, xd, preferred_element_type=jnp.float32)
        cnt = cnt_ref[...]
        sums = jnp.dot(z, wc_ref[...], preferred_element_type=jnp.float32)
        sums = sums + cnt * bc_ref[...]
        pooled = sums / jnp.maximum(cnt, 1.0)
        o_ref[...] = (jnp.dot(pooled, wl_ref[...],
                              preferred_element_type=jnp.float32)
                      + bl_ref[...])

    return pl.pallas_call(
        body,
        out_shape=jax.ShapeDtypeStruct((NG, NUM_CLASSES), jnp.float32),
    )(cp, x, dinv_col, cnt, Wc, bc, Wl, bl)


def kernel(x, edge_index, batch, W_conv, b_conv, W_lin, b_lin):
    er3 = jnp.pad(edge_index, ((0, 0), (0, ROWS * 128 - E)),
                  constant_values=NPAD - 1).reshape(2, ROWS, 128)
    batch_p = jnp.pad(batch, (0, NPAD - N))

    deg_part = _sc_degree(er3)
    dinv = _tc_dinv(deg_part.reshape(NC, NPAD // 128, 128))
    dinv_flat = dinv.reshape(NPAD)
    cp = _sc_coef(er3, dinv_flat, batch_p)
    cnt = _tc_counts(batch.reshape(1, N))
    out = _tc_head(cp.reshape(NC, NG, N), x,
                   dinv_flat[:N].reshape(N, 1), cnt,
                   W_conv, b_conv.reshape(1, HIDDEN),
                   W_lin, b_lin.reshape(1, NUM_CLASSES))
    return out


# speedup vs baseline: 1.0393x; 1.0393x over previous
import functools

import jax
import jax.numpy as jnp
from jax import lax
from jax.experimental import pallas as pl
from jax.experimental.pallas import tpu as pltpu
from jax.experimental.pallas import tpu_sc as plsc

N = 10000
E = 320000
D_IN = 128
HIDDEN = 128
NUM_CLASSES = 10
NG = 64

NC = 2
NS = 16
TILES = NC * NS
RPT = 80
ROWS = TILES * RPT
NPAD = 10240
SPT = 3
CH = RPT + SPT
CN = NG * N
NW = 10112


def _sc_mesh():
    return plsc.VectorSubcoreMesh(core_axis_name="c", subcore_axis_name="s")


def _sc_degree(er3):
    @functools.partial(
        pl.kernel,
        mesh=_sc_mesh(),
        compiler_params=pltpu.CompilerParams(needs_layout_passes=False),
        out_type=jax.ShapeDtypeStruct((NC * NPAD,), jnp.float32),
        scratch_types=[
            pltpu.VMEM((RPT, 128), jnp.int32),
            pltpu.VMEM((128,), jnp.float32),
            pltpu.VMEM((640,), jnp.float32),
            pltpu.VMEM_SHARED((NPAD,), jnp.float32),
            pltpu.SemaphoreType.DMA,
            pltpu.SemaphoreType.DMA,
        ],
    )
    def k(er_hbm, out_hbm, dst_v, ones_v, zero_v, deg_sh, sem_in, sem_s):
        c = lax.axis_index("c")
        s = lax.axis_index("s")
        wid = s * NC + c
        nrows = jnp.clip(E // 128 - wid * RPT, 0, RPT)

        pltpu.async_copy(er_hbm.at[1, pl.ds(wid * RPT, RPT), :], dst_v,
                         sem_in)

        def fbody(i, _):
            ones_v[pl.ds(i * 16, 16)] = jnp.full((16,), 1.0, jnp.float32)
            zero_v[pl.ds(i * 16, 16)] = jnp.zeros((16,), jnp.float32)
            return 0

        lax.fori_loop(0, 8, fbody, 0)

        def zbody(i, _):
            zero_v[pl.ds(128 + i * 16, 16)] = jnp.zeros((16,), jnp.float32)
            return 0

        lax.fori_loop(0, (640 - 128) // 16, zbody, 0)
        pltpu.sync_copy(zero_v, deg_sh.at[pl.ds(s * 640, 640)])
        pltpu.make_async_copy(er_hbm.at[1, pl.ds(wid * RPT, RPT), :], dst_v,
                              sem_in).wait()
        plsc.subcore_barrier()

        def sbody(j, _):
            pltpu.async_copy(ones_v, deg_sh.at[dst_v.at[j]], sem_s, add=True)
            return 0

        lax.fori_loop(0, nrows, sbody, 0)

        def dbody(j, _):
            pltpu.make_async_copy(ones_v, deg_sh.at[dst_v.at[0]],
                                  sem_s).wait()
            return 0

        lax.fori_loop(0, nrows, dbody, 0)
        plsc.subcore_barrier()
        pltpu.sync_copy(deg_sh.at[pl.ds(s * 640, 640)], zero_v)
        pltpu.sync_copy(zero_v, out_hbm.at[pl.ds(c * NPAD + s * 640, 640)])

    return k(er3)


def _tc_dinv(deg_part):
    def body(d_ref, o_ref):
        deg = d_ref[0] + d_ref[1] + 1.0
        dinv = lax.rsqrt(deg)
        r = lax.broadcasted_iota(jnp.int32, (NPAD // 128, 128), 0)
        l = lax.broadcasted_iota(jnp.int32, (NPAD // 128, 128), 1)
        o_ref[...] = jnp.where(r * 128 + l < N, dinv, 0.0)

    return pl.pallas_call(
        body,
        out_shape=jax.ShapeDtypeStruct((NPAD // 128, 128), jnp.float32),
    )(deg_part)


def _sc_coef(er3, dinv_p, batch_p):
    ZS = CN // NS

    @functools.partial(
        pl.kernel,
        mesh=_sc_mesh(),
        compiler_params=pltpu.CompilerParams(needs_layout_passes=False),
        out_type=jax.ShapeDtypeStruct((NC * CN,), jnp.float32),
        scratch_types=[
            pltpu.VMEM((RPT, 128), jnp.int32),
            pltpu.VMEM((RPT, 128), jnp.int32),
            pltpu.VMEM((NPAD,), jnp.float32),
            pltpu.VMEM((NPAD,), jnp.int32),
            pltpu.VMEM((CH, 128), jnp.int32),
            pltpu.VMEM((CH, 128), jnp.float32),
            pltpu.VMEM((8000,), jnp.float32),
            pltpu.VMEM((8000,), jnp.float32),
            pltpu.VMEM_SHARED((CN,), jnp.float32),
            pltpu.SemaphoreType.DMA,
            pltpu.SemaphoreType.DMA,
            pltpu.SemaphoreType.DMA,
            pltpu.SemaphoreType.DMA,
            pltpu.SemaphoreType.DMA,
            pltpu.SemaphoreType.DMA,
        ],
    )
    def k(er_hbm, dinv_hbm, batch_hbm, out_hbm,
          src_v, dst_v, dinv_v, batch_v, idx_v, val_v, buf0, buf1,
          c_sh, sem_in, sem_s, sem_z, sem_a1, sem_b0, sem_b1):
        c = lax.axis_index("c")
        s = lax.axis_index("s")
        wid = s * NC + c
        nrows = jnp.clip(E // 128 - wid * RPT, 0, RPT)
        base = wid * (SPT * 128)
        nself = jnp.where(base < N, SPT, 0)

        pltpu.async_copy(er_hbm.at[0, pl.ds(wid * RPT, RPT), :], src_v,
                         sem_in)
        pltpu.async_copy(er_hbm.at[1, pl.ds(wid * RPT, RPT), :], dst_v,
                         sem_in)
        pltpu.async_copy(dinv_hbm, dinv_v, sem_in)
        pltpu.async_copy(batch_hbm, batch_v, sem_in)

        def zbody(i, _):
            buf0[pl.ds(i * 16, 16)] = jnp.zeros((16,), jnp.float32)
            return 0

        lax.fori_loop(0, 8000 // 16, zbody, 0)

        def zcopy(i, _):
            pltpu.async_copy(buf0, c_sh.at[pl.ds(s * ZS + i * 8000, 8000)],
                             sem_z)
            return 0

        lax.fori_loop(0, ZS // 8000, zcopy, 0)

        def zdrain(i, _):
            pltpu.make_async_copy(buf0, c_sh.at[pl.ds(s * ZS, 8000)],
                                  sem_z).wait()
            return 0

        lax.fori_loop(0, ZS // 8000, zdrain, 0)

        pltpu.make_async_copy(er_hbm.at[0, pl.ds(wid * RPT, RPT), :], src_v,
                              sem_in).wait()
        pltpu.make_async_copy(er_hbm.at[1, pl.ds(wid * RPT, RPT), :], dst_v,
                              sem_in).wait()
        pltpu.make_async_copy(dinv_hbm, dinv_v, sem_in).wait()
        pltpu.make_async_copy(batch_hbm, batch_v, sem_in).wait()
        plsc.subcore_barrier()

        def cbody(j, _):
            for kk in range(8):
                sl = pl.ds(kk * 16, 16)
                d16 = dst_v[j, sl]
                s16 = src_v[j, sl]
                b16 = plsc.load_gather(batch_v, [d16])
                dv = plsc.load_gather(dinv_v, [d16])
                val_v[j, sl] = dv
                idx_v[j, sl] = b16 * N + s16
            pltpu.async_copy(val_v.at[j], c_sh.at[idx_v.at[j]], sem_s,
                             add=True)
            return 0

        lax.fori_loop(0, nrows, cbody, 0)

        @pl.when(base < N)
        def _():
            for jj in range(SPT):
                for kk in range(8):
                    sl = pl.ds(kk * 16, 16)
                    i16 = (base + jj * 128 + kk * 16
                           + lax.broadcasted_iota(jnp.int32, (16,), 0))
                    i16 = jnp.where(i16 >= NPAD, i16 - 128, i16)
                    b16 = plsc.load_gather(batch_v, [i16])
                    dv = plsc.load_gather(dinv_v, [i16])
                    val_v[RPT + jj, sl] = dv
                    idx_v[RPT + jj, sl] = b16 * N + i16
                pltpu.async_copy(val_v.at[RPT + jj],
                                 c_sh.at[idx_v.at[RPT + jj]],
                                 sem_s, add=True)

        def dbody(j, _):
            pltpu.make_async_copy(val_v.at[0], c_sh.at[idx_v.at[0]],
                                  sem_s).wait()
            return 0

        lax.fori_loop(0, nrows + nself, dbody, 0)
        plsc.subcore_barrier()

        bufs = (buf0, buf1)
        sa = (sem_in, sem_a1)
        sb = (sem_b0, sem_b1)

        def rd(i):
            return (c_sh.at[pl.ds(s * ZS + i * 8000, 8000)], bufs[i % 2],
                    sa[i % 2])

        def wr(i):
            return (bufs[i % 2],
                    out_hbm.at[pl.ds(c * CN + s * ZS + i * 8000, 8000)],
                    sb[i % 2])

        nchunk = ZS // 8000
        pltpu.async_copy(*rd(0))
        pltpu.async_copy(*rd(1))
        for i in range(nchunk):
            pltpu.make_async_copy(*rd(i)).wait()
            pltpu.async_copy(*wr(i))
            if i + 2 < nchunk:
                pltpu.make_async_copy(*wr(i)).wait()
                pltpu.async_copy(*rd(i + 2))
        pltpu.make_async_copy(*wr(nchunk - 2)).wait()
        pltpu.make_async_copy(*wr(nchunk - 1)).wait()

    return k(er3, dinv_p, batch_p)


def _tc_counts(batch_row):
    def body(b_ref, o_ref):
        g = lax.broadcasted_iota(jnp.int32, (NG, N), 0)
        o_ref[...] = jnp.sum(jnp.where(b_ref[...] == g, 1.0, 0.0), axis=1,
                             keepdims=True)

    return pl.pallas_call(
        body,
        out_shape=jax.ShapeDtypeStruct((NG, 1), jnp.float32),
    )(batch_row)


def _tc_head(cp, x, dinv_col, cnt, Wc, bc, Wl, bl):
    def body(cp_ref, x_ref, dinv_ref, cnt_ref, wc_ref, bc_ref, wl_ref, bl_ref,
             o_ref):
        C = cp_ref[0] + cp_ref[1]
        xd = x_ref[...] * dinv_ref[...]
        z = jnp.dot(C, xd, preferred_element_type=jnp.float32)
        cnt = cnt_ref[...]
        sums = jnp.dot(z, wc_ref[...], preferred_element_type=jnp.float32)
        sums = sums + cnt * bc_ref[...]
        pooled = sums / jnp.maximum(cnt, 1.0)
        o_ref[...] = (jnp.dot(pooled, wl_ref[...],
                              preferred_element_type=jnp.float32)
                      + bl_ref[...])

    return pl.pallas_call(
        body,
        out_shape=jax.ShapeDtypeStruct((NG, NUM_CLASSES), jnp.float32),
    )(cp, x, dinv_col, cnt, Wc, bc, Wl, bl)


def kernel(x, edge_index, batch, W_conv, b_conv, W_lin, b_lin):
    er3 = jnp.pad(edge_index, ((0, 0), (0, ROWS * 128 - E)),
                  constant_values=NPAD - 1).reshape(2, ROWS, 128)
    batch_p = jnp.pad(batch, (0, NPAD - N))

    deg_part = _sc_degree(er3)
    dinv = _tc_dinv(deg_part.reshape(NC, NPAD // 128, 128))
    dinv_flat = dinv.reshape(NPAD)
    cp = _sc_coef(er3, dinv_flat, batch_p)
    cnt = _tc_counts(batch.reshape(1, N))
    out = _tc_head(cp.reshape(NC, NG, N), x,
                   dinv_flat[:N].reshape(N, 1), cnt,
                   W_conv, b_conv.reshape(1, HIDDEN),
                   W_lin, b_lin.reshape(1, NUM_CLASSES))
    return out


# speedup vs baseline: 1.0427x; 1.0033x over previous
import functools

import jax
import jax.numpy as jnp
from jax import lax
from jax.experimental import pallas as pl
from jax.experimental.pallas import tpu as pltpu
from jax.experimental.pallas import tpu_sc as plsc

N = 10000
E = 320000
D_IN = 128
HIDDEN = 128
NUM_CLASSES = 10
NG = 64

NC = 2
NS = 16
TILES = NC * NS
RPT = 80
ROWS = TILES * RPT
NPAD = 10240
SPT = 3
CH = RPT + SPT
CN = NG * N


def _sc_mesh():
    return plsc.VectorSubcoreMesh(core_axis_name="c", subcore_axis_name="s")


def _sc_degree(er3):
    @functools.partial(
        pl.kernel,
        mesh=_sc_mesh(),
        compiler_params=pltpu.CompilerParams(needs_layout_passes=False),
        out_type=jax.ShapeDtypeStruct((NC * NPAD,), jnp.float32),
        scratch_types=[
            pltpu.VMEM((RPT, 128), jnp.int32),
            pltpu.VMEM((128,), jnp.float32),
            pltpu.VMEM((640,), jnp.float32),
            pltpu.VMEM_SHARED((NPAD,), jnp.float32),
            pltpu.SemaphoreType.DMA,
            pltpu.SemaphoreType.DMA,
        ],
    )
    def k(er_hbm, out_hbm, dst_v, ones_v, zero_v, deg_sh, sem_in, sem_s):
        c = lax.axis_index("c")
        s = lax.axis_index("s")
        wid = s * NC + c
        nrows = jnp.clip(E // 128 - wid * RPT, 0, RPT)

        pltpu.async_copy(er_hbm.at[1, pl.ds(wid * RPT, RPT), :], dst_v,
                         sem_in)

        def fbody(i, _):
            ones_v[pl.ds(i * 16, 16)] = jnp.full((16,), 1.0, jnp.float32)
            zero_v[pl.ds(i * 16, 16)] = jnp.zeros((16,), jnp.float32)
            return 0

        lax.fori_loop(0, 8, fbody, 0)

        def zbody(i, _):
            zero_v[pl.ds(128 + i * 16, 16)] = jnp.zeros((16,), jnp.float32)
            return 0

        lax.fori_loop(0, (640 - 128) // 16, zbody, 0)
        pltpu.sync_copy(zero_v, deg_sh.at[pl.ds(s * 640, 640)])
        pltpu.make_async_copy(er_hbm.at[1, pl.ds(wid * RPT, RPT), :], dst_v,
                              sem_in).wait()
        plsc.subcore_barrier()

        def sbody(j, _):
            pltpu.async_copy(ones_v, deg_sh.at[dst_v.at[j]], sem_s, add=True)
            return 0

        lax.fori_loop(0, nrows, sbody, 0)

        def dbody(j, _):
            pltpu.make_async_copy(ones_v, deg_sh.at[dst_v.at[0]],
                                  sem_s).wait()
            return 0

        lax.fori_loop(0, nrows, dbody, 0)
        plsc.subcore_barrier()
        pltpu.sync_copy(deg_sh.at[pl.ds(s * 640, 640)], zero_v)
        pltpu.sync_copy(zero_v, out_hbm.at[pl.ds(c * NPAD + s * 640, 640)])

    return k(er3)


def _tc_dinv(deg_part):
    def body(d_ref, o_ref):
        deg = d_ref[0] + d_ref[1] + 1.0
        dinv = lax.rsqrt(deg)
        r = lax.broadcasted_iota(jnp.int32, (NPAD // 128, 128), 0)
        l = lax.broadcasted_iota(jnp.int32, (NPAD // 128, 128), 1)
        o_ref[...] = jnp.where(r * 128 + l < N, dinv, 0.0)

    return pl.pallas_call(
        body,
        out_shape=jax.ShapeDtypeStruct((NPAD // 128, 128), jnp.float32),
    )(deg_part)


def _sc_coef(er3, dinv_p, batch_p):
    ZS = CN // NS

    @functools.partial(
        pl.kernel,
        mesh=_sc_mesh(),
        compiler_params=pltpu.CompilerParams(needs_layout_passes=False),
        out_type=jax.ShapeDtypeStruct((NC * CN,), jnp.float32),
        scratch_types=[
            pltpu.VMEM((RPT, 128), jnp.int32),
            pltpu.VMEM((RPT, 128), jnp.int32),
            pltpu.VMEM((NPAD,), jnp.float32),
            pltpu.VMEM((NPAD,), jnp.int32),
            pltpu.VMEM((CH, 128), jnp.int32),
            pltpu.VMEM((CH, 128), jnp.float32),
            pltpu.VMEM((8000,), jnp.float32),
            pltpu.VMEM((8000,), jnp.float32),
            pltpu.VMEM_SHARED((CN,), jnp.float32),
            pltpu.SemaphoreType.DMA,
            pltpu.SemaphoreType.DMA,
            pltpu.SemaphoreType.DMA,
            pltpu.SemaphoreType.DMA,
            pltpu.SemaphoreType.DMA,
            pltpu.SemaphoreType.DMA,
        ],
    )
    def k(er_hbm, dinv_hbm, batch_hbm, out_hbm,
          src_v, dst_v, dinv_v, batch_v, idx_v, val_v, buf0, buf1,
          c_sh, sem_in, sem_s, sem_z, sem_a1, sem_b0, sem_b1):
        c = lax.axis_index("c")
        s = lax.axis_index("s")
        wid = s * NC + c
        nrows = jnp.clip(E // 128 - wid * RPT, 0, RPT)
        base = wid * (SPT * 128)
        nself = jnp.where(base < N, SPT, 0)

        pltpu.async_copy(er_hbm.at[0, pl.ds(wid * RPT, RPT), :], src_v,
                         sem_in)
        pltpu.async_copy(er_hbm.at[1, pl.ds(wid * RPT, RPT), :], dst_v,
                         sem_in)
        pltpu.async_copy(dinv_hbm, dinv_v, sem_in)
        pltpu.async_copy(batch_hbm, batch_v, sem_in)

        def zbody(i, _):
            buf0[pl.ds(i * 16, 16)] = jnp.zeros((16,), jnp.float32)
            return 0

        lax.fori_loop(0, 8000 // 16, zbody, 0)

        def zcopy(i, _):
            pltpu.async_copy(buf0, c_sh.at[pl.ds(s * ZS + i * 8000, 8000)],
                             sem_z)
            return 0

        lax.fori_loop(0, ZS // 8000, zcopy, 0)

        def zdrain(i, _):
            pltpu.make_async_copy(buf0, c_sh.at[pl.ds(s * ZS, 8000)],
                                  sem_z).wait()
            return 0

        lax.fori_loop(0, ZS // 8000, zdrain, 0)

        pltpu.make_async_copy(er_hbm.at[0, pl.ds(wid * RPT, RPT), :], src_v,
                              sem_in).wait()
        pltpu.make_async_copy(er_hbm.at[1, pl.ds(wid * RPT, RPT), :], dst_v,
                              sem_in).wait()
        pltpu.make_async_copy(dinv_hbm, dinv_v, sem_in).wait()
        pltpu.make_async_copy(batch_hbm, batch_v, sem_in).wait()
        plsc.subcore_barrier()

        def cbody(j, _):
            for kk in range(8):
                sl = pl.ds(kk * 16, 16)
                d16 = dst_v[j, sl]
                s16 = src_v[j, sl]
                b16 = plsc.load_gather(batch_v, [d16])
                dv = plsc.load_gather(dinv_v, [d16])
                val_v[j, sl] = dv
                idx_v[j, sl] = b16 * N + s16
            pltpu.async_copy(val_v.at[j], c_sh.at[idx_v.at[j]], sem_s,
                             add=True)
            return 0

        lax.fori_loop(0, nrows, cbody, 0)

        @pl.when(base < N)
        def _():
            for jj in range(SPT):
                for kk in range(8):
                    sl = pl.ds(kk * 16, 16)
                    i16 = (base + jj * 128 + kk * 16
                           + lax.broadcasted_iota(jnp.int32, (16,), 0))
                    i16 = jnp.where(i16 >= NPAD, i16 - 128, i16)
                    b16 = plsc.load_gather(batch_v, [i16])
                    dv = plsc.load_gather(dinv_v, [i16])
                    val_v[RPT + jj, sl] = dv
                    idx_v[RPT + jj, sl] = b16 * N + i16
                pltpu.async_copy(val_v.at[RPT + jj],
                                 c_sh.at[idx_v.at[RPT + jj]],
                                 sem_s, add=True)

        def dbody(j, _):
            pltpu.make_async_copy(val_v.at[0], c_sh.at[idx_v.at[0]],
                                  sem_s).wait()
            return 0

        lax.fori_loop(0, nrows + nself, dbody, 0)
        plsc.subcore_barrier()

        bufs = (buf0, buf1)
        sa = (sem_in, sem_a1)
        sb = (sem_b0, sem_b1)

        def rd(i):
            return (c_sh.at[pl.ds(s * ZS + i * 8000, 8000)], bufs[i % 2],
                    sa[i % 2])

        def wr(i):
            return (bufs[i % 2],
                    out_hbm.at[pl.ds(c * CN + s * ZS + i * 8000, 8000)],
                    sb[i % 2])

        nchunk = ZS // 8000
        pltpu.async_copy(*rd(0))
        pltpu.async_copy(*rd(1))
        for i in range(nchunk):
            pltpu.make_async_copy(*rd(i)).wait()
            pltpu.async_copy(*wr(i))
            if i + 2 < nchunk:
                pltpu.make_async_copy(*wr(i)).wait()
                pltpu.async_copy(*rd(i + 2))
        pltpu.make_async_copy(*wr(nchunk - 2)).wait()
        pltpu.make_async_copy(*wr(nchunk - 1)).wait()

    return k(er3, dinv_p, batch_p)


def _tc_counts(batch_row):
    def body(b_ref, o_ref):
        g = lax.broadcasted_iota(jnp.int32, (NG, N), 0)
        o_ref[...] = jnp.sum(jnp.where(b_ref[...] == g, 1.0, 0.0), axis=1,
                             keepdims=True)

    return pl.pallas_call(
        body,
        out_shape=jax.ShapeDtypeStruct((NG, 1), jnp.float32),
    )(batch_row)


def _tc_head(cp, x, dinv_col, cnt, Wc, bc, Wl, bl):
    def body(cp_ref, x_ref, dinv_ref, cnt_ref, wc_ref, bc_ref, wl_ref, bl_ref,
             o_ref):
        C = cp_ref[0] + cp_ref[1]
        xd = x_ref[...] * dinv_ref[...]
        z = jnp.dot(C, xd, preferred_element_type=jnp.float32)
        cnt = cnt_ref[...]
        sums = jnp.dot(z, wc_ref[...], preferred_element_type=jnp.float32)
        sums = sums + cnt * bc_ref[...]
        pooled = sums / jnp.maximum(cnt, 1.0)
        o_ref[...] = (jnp.dot(pooled, wl_ref[...],
                              preferred_element_type=jnp.float32)
                      + bl_ref[...])

    return pl.pallas_call(
        body,
        out_shape=jax.ShapeDtypeStruct((NG, NUM_CLASSES), jnp.float32),
    )(cp, x, dinv_col, cnt, Wc, bc, Wl, bl)


def kernel(x, edge_index, batch, W_conv, b_conv, W_lin, b_lin):
    er3 = jnp.pad(edge_index, ((0, 0), (0, ROWS * 128 - E)),
                  constant_values=NPAD - 1).reshape(2, ROWS, 128)
    batch_p = jnp.pad(batch, (0, NPAD - N))

    deg_part = _sc_degree(er3)
    dinv = _tc_dinv(deg_part.reshape(NC, NPAD // 128, 128))
    dinv_flat = dinv.reshape(NPAD)
    cp = _sc_coef(er3, dinv_flat, batch_p)
    cnt = _tc_counts(batch.reshape(1, N))
    out = _tc_head(cp.reshape(NC, NG, N), x,
                   dinv_flat[:N].reshape(N, 1), cnt,
                   W_conv, b_conv.reshape(1, HIDDEN),
                   W_lin, b_lin.reshape(1, NUM_CLASSES))
    return out
